# baseline (speedup 1.0000x reference)
import functools

import jax
import jax.numpy as jnp
from jax.experimental import pallas as pl
from jax.experimental.pallas import tpu as pltpu


NUM_Q = 4
K = 1024
D = 32
BLOCK = 256
KC = 256


def _rvq_body(x_ref, cb_ref, out_ref, loss_ref, *, n_tokens):
    pid = pl.program_id(0)
    r = x_ref[...]
    qsum = jnp.zeros_like(r)
    acc = jnp.zeros((1, 1), jnp.float32)
    chunk_iota = jax.lax.broadcasted_iota(jnp.int32, (1, KC), 1)
    nchunks = K // KC

    for i in range(NUM_Q):
        r2 = jnp.sum(r * r, axis=1, keepdims=True)
        r2b = (2.0 * r).astype(jnp.bfloat16)

        def argmin_step(c, carry, i=i, r2=r2, r2b=r2b):
            best_val, best_idx = carry
            Wc = cb_ref[i, pl.ds(c * KC, KC), :]
            w2 = jnp.sum(Wc * Wc, axis=1)
            scores = jax.lax.dot_general(
                r2b, Wc.astype(jnp.bfloat16), (((1,), (1,)), ((), ())),
                preferred_element_type=jnp.float32)
            dist = (r2 - scores) + w2[None, :]
            cmin = jnp.min(dist, axis=1, keepdims=True)
            cidx = jnp.min(
                jnp.where(dist <= cmin, chunk_iota + c * KC, K),
                axis=1, keepdims=True)
            better = cmin < best_val
            return (jnp.where(better, cmin, best_val),
                    jnp.where(better, cidx, best_idx))

        best_val, best_idx = jax.lax.fori_loop(
            0, nchunks, argmin_step,
            (jnp.full((BLOCK, 1), jnp.inf, jnp.float32),
             jnp.zeros((BLOCK, 1), jnp.int32)))

        def gather_step(c, q, i=i, best_idx=best_idx):
            Wc = cb_ref[i, pl.ds(c * KC, KC), :]
            onehot = (chunk_iota + c * KC == best_idx).astype(jnp.float32)
            return q + jax.lax.dot_general(
                onehot, Wc, (((1,), (0,)), ((), ())),
                precision=jax.lax.Precision.HIGHEST,
                preferred_element_type=jnp.float32)

        q = jax.lax.fori_loop(
            0, nchunks, gather_step, jnp.zeros((BLOCK, D), jnp.float32))
        r = r - q
        qsum = qsum + q
        acc += jnp.sum(r * r, axis=(0, 1), keepdims=True)

    out_ref[...] = qsum

    @pl.when(pid == 0)
    def _():
        loss_ref[...] = jnp.zeros((1, 1), jnp.float32)

    loss_ref[...] += acc * (1.25 / (n_tokens * D))


def kernel(x, codebooks):
    n = x.shape[0]
    grid = n // BLOCK
    out, loss = pl.pallas_call(
        functools.partial(_rvq_body, n_tokens=n),
        grid=(grid,),
        in_specs=[
            pl.BlockSpec((BLOCK, D), lambda i: (i, 0)),
            pl.BlockSpec((NUM_Q, K, D), lambda i: (0, 0, 0)),
        ],
        out_specs=[
            pl.BlockSpec((BLOCK, D), lambda i: (i, 0)),
            pl.BlockSpec((1, 1), lambda i: (0, 0)),
        ],
        out_shape=[
            jax.ShapeDtypeStruct((n, D), jnp.float32),
            jax.ShapeDtypeStruct((1, 1), jnp.float32),
        ],
        compiler_params=pltpu.CompilerParams(
            dimension_semantics=("arbitrary",),
            vmem_limit_bytes=100 * 1024 * 1024,
        ),
    )(x, codebooks)
    return out, loss[0, 0]


# speedup vs baseline: 6.1471x; 6.1471x over previous
import functools

import jax
import jax.numpy as jnp
from jax.experimental import pallas as pl
from jax.experimental.pallas import tpu as pltpu


NUM_Q = 4
K = 1024
D = 32
BLOCK = 512


def _rvq_body(xt_ref, cb_ref, cbt_ref, out_ref, loss_ref, *, n_tokens):
    pid = pl.program_id(0)
    r = xt_ref[...]
    qsum = jnp.zeros_like(r)
    acc = jnp.zeros((1, 1), jnp.float32)
    iota_k = jax.lax.broadcasted_iota(jnp.int32, (K, 1), 0)

    for i in range(NUM_Q):
        Wc = cb_ref[i]
        Wt = cbt_ref[i]
        w2 = jnp.sum(Wc * Wc, axis=1, keepdims=True)
        r2 = jnp.sum(r * r, axis=0, keepdims=True)
        r2b = (2.0 * r).astype(jnp.bfloat16)
        scores = jax.lax.dot_general(
            Wc.astype(jnp.bfloat16), r2b, (((1,), (0,)), ((), ())),
            preferred_element_type=jnp.float32)
        dist = (r2 - scores) + w2
        cmin = jnp.min(dist, axis=0, keepdims=True)
        idx = jnp.min(
            jnp.where(dist <= cmin, iota_k, K),
            axis=0, keepdims=True)
        onehot = (iota_k == idx).astype(jnp.bfloat16)
        hi = Wt.astype(jnp.bfloat16)
        rem = Wt - hi.astype(jnp.float32)
        mid = rem.astype(jnp.bfloat16)
        lo = (rem - mid.astype(jnp.float32)).astype(jnp.bfloat16)
        wsplit = jnp.concatenate([hi, mid, lo], axis=0)
        q3 = jax.lax.dot_general(
            wsplit, onehot, (((1,), (0,)), ((), ())),
            preferred_element_type=jnp.float32)
        q = (q3[0:D, :] + q3[D:2 * D, :]) + q3[2 * D:3 * D, :]
        r = r - q
        qsum = qsum + q
        acc += jnp.sum(r * r, axis=(0, 1), keepdims=True)

    out_ref[...] = qsum

    @pl.when(pid == 0)
    def _():
        loss_ref[...] = jnp.zeros((1, 1), jnp.float32)

    loss_ref[...] += acc * (1.25 / (n_tokens * D))


def kernel(x, codebooks):
    n = x.shape[0]
    grid = n // BLOCK
    xt = x.T
    cbt = jnp.swapaxes(codebooks, 1, 2)
    out_t, loss = pl.pallas_call(
        functools.partial(_rvq_body, n_tokens=n),
        grid=(grid,),
        in_specs=[
            pl.BlockSpec((D, BLOCK), lambda i: (0, i)),
            pl.BlockSpec((NUM_Q, K, D), lambda i: (0, 0, 0)),
            pl.BlockSpec((NUM_Q, D, K), lambda i: (0, 0, 0)),
        ],
        out_specs=[
            pl.BlockSpec((D, BLOCK), lambda i: (0, i)),
            pl.BlockSpec((1, 1), lambda i: (0, 0)),
        ],
        out_shape=[
            jax.ShapeDtypeStruct((D, n), jnp.float32),
            jax.ShapeDtypeStruct((1, 1), jnp.float32),
        ],
        compiler_params=pltpu.CompilerParams(
            dimension_semantics=("arbitrary",),
            vmem_limit_bytes=100 * 1024 * 1024,
        ),
    )(xt, codebooks, cbt)
    return out_t.T, loss[0, 0]


# speedup vs baseline: 8.0537x; 1.3102x over previous
import functools

import jax
import jax.numpy as jnp
from jax.experimental import pallas as pl
from jax.experimental.pallas import tpu as pltpu


NUM_Q = 4
K = 1024
D = 32
BLOCK = 1024


def _rvq_body(xt_ref, cb_ref, cbt_ref, out_ref, loss_ref, *, n_tokens):
    pid = pl.program_id(0)
    r = xt_ref[...]
    qsum = jnp.zeros_like(r)
    acc = jnp.zeros((1, 1), jnp.float32)
    iota_k = jax.lax.broadcasted_iota(jnp.int32, (K, 1), 0)

    for i in range(NUM_Q):
        Wc = cb_ref[i]
        Wt = cbt_ref[i]
        w2 = jnp.sum(Wc * Wc, axis=1, keepdims=True)
        r2 = jnp.sum(r * r, axis=0, keepdims=True)
        r2b = (2.0 * r).astype(jnp.bfloat16)
        scores = jax.lax.dot_general(
            Wc.astype(jnp.bfloat16), r2b, (((1,), (0,)), ((), ())),
            preferred_element_type=jnp.float32)
        dist = (r2 - scores) + w2
        cmin = jnp.min(dist, axis=0, keepdims=True)
        idx = jnp.min(
            jnp.where(dist <= cmin, iota_k, K),
            axis=0, keepdims=True)
        onehot = (iota_k == idx).astype(jnp.bfloat16)
        hi = Wt.astype(jnp.bfloat16)
        rem = Wt - hi.astype(jnp.float32)
        mid = rem.astype(jnp.bfloat16)
        lo = (rem - mid.astype(jnp.float32)).astype(jnp.bfloat16)
        wsplit = jnp.concatenate([hi, mid, lo], axis=0)
        q3 = jax.lax.dot_general(
            wsplit, onehot, (((1,), (0,)), ((), ())),
            preferred_element_type=jnp.float32)
        q = (q3[0:D, :] + q3[D:2 * D, :]) + q3[2 * D:3 * D, :]
        r = r - q
        qsum = qsum + q
        acc += jnp.sum(r * r, axis=(0, 1), keepdims=True)

    out_ref[...] = qsum

    @pl.when(pid == 0)
    def _():
        loss_ref[...] = jnp.zeros((1, 1), jnp.float32)

    loss_ref[...] += acc * (1.25 / (n_tokens * D))


def kernel(x, codebooks):
    n = x.shape[0]
    grid = n // BLOCK
    xt = x.T
    cbt = jnp.swapaxes(codebooks, 1, 2)
    out_t, loss = pl.pallas_call(
        functools.partial(_rvq_body, n_tokens=n),
        grid=(grid,),
        in_specs=[
            pl.BlockSpec((D, BLOCK), lambda i: (0, i)),
            pl.BlockSpec((NUM_Q, K, D), lambda i: (0, 0, 0)),
            pl.BlockSpec((NUM_Q, D, K), lambda i: (0, 0, 0)),
        ],
        out_specs=[
            pl.BlockSpec((D, BLOCK), lambda i: (0, i)),
            pl.BlockSpec((1, 1), lambda i: (0, 0)),
        ],
        out_shape=[
            jax.ShapeDtypeStruct((D, n), jnp.float32),
            jax.ShapeDtypeStruct((1, 1), jnp.float32),
        ],
        compiler_params=pltpu.CompilerParams(
            dimension_semantics=("arbitrary",),
            vmem_limit_bytes=100 * 1024 * 1024,
        ),
    )(xt, codebooks, cbt)
    return out_t.T, loss[0, 0]


# speedup vs baseline: 8.8852x; 1.1033x over previous
import functools

import jax
import jax.numpy as jnp
from jax.experimental import pallas as pl
from jax.experimental.pallas import tpu as pltpu


NUM_Q = 4
K = 1024
D = 32
BLOCK = 2048


def _rvq_body(xt_ref, cb_ref, cbt_ref, out_ref, loss_ref, *, n_tokens):
    pid = pl.program_id(0)
    r = xt_ref[...]
    qsum = jnp.zeros_like(r)
    acc = jnp.zeros((1, 1), jnp.float32)
    iota_k = jax.lax.broadcasted_iota(jnp.int32, (K, 1), 0)

    for i in range(NUM_Q):
        Wc = cb_ref[i]
        Wt = cbt_ref[i]
        w2 = jnp.sum(Wc * Wc, axis=1, keepdims=True)
        r2 = jnp.sum(r * r, axis=0, keepdims=True)
        r2b = (2.0 * r).astype(jnp.bfloat16)
        scores = jax.lax.dot_general(
            Wc.astype(jnp.bfloat16), r2b, (((1,), (0,)), ((), ())),
            preferred_element_type=jnp.float32)
        dist = (r2 - scores) + w2
        cmin = jnp.min(dist, axis=0, keepdims=True)
        idx = jnp.min(
            jnp.where(dist <= cmin, iota_k, K),
            axis=0, keepdims=True)
        onehot = (iota_k == idx).astype(jnp.bfloat16)
        hi = Wt.astype(jnp.bfloat16)
        rem = Wt - hi.astype(jnp.float32)
        mid = rem.astype(jnp.bfloat16)
        lo = (rem - mid.astype(jnp.float32)).astype(jnp.bfloat16)
        wsplit = jnp.concatenate([hi, mid, lo], axis=0)
        q3 = jax.lax.dot_general(
            wsplit, onehot, (((1,), (0,)), ((), ())),
            preferred_element_type=jnp.float32)
        q = (q3[0:D, :] + q3[D:2 * D, :]) + q3[2 * D:3 * D, :]
        r = r - q
        qsum = qsum + q
        acc += jnp.sum(r * r, axis=(0, 1), keepdims=True)

    out_ref[...] = qsum

    @pl.when(pid == 0)
    def _():
        loss_ref[...] = jnp.zeros((1, 1), jnp.float32)

    loss_ref[...] += acc * (1.25 / (n_tokens * D))


def kernel(x, codebooks):
    n = x.shape[0]
    grid = n // BLOCK
    xt = x.T
    cbt = jnp.swapaxes(codebooks, 1, 2)
    out_t, loss = pl.pallas_call(
        functools.partial(_rvq_body, n_tokens=n),
        grid=(grid,),
        in_specs=[
            pl.BlockSpec((D, BLOCK), lambda i: (0, i)),
            pl.BlockSpec((NUM_Q, K, D), lambda i: (0, 0, 0)),
            pl.BlockSpec((NUM_Q, D, K), lambda i: (0, 0, 0)),
        ],
        out_specs=[
            pl.BlockSpec((D, BLOCK), lambda i: (0, i)),
            pl.BlockSpec((1, 1), lambda i: (0, 0)),
        ],
        out_shape=[
            jax.ShapeDtypeStruct((D, n), jnp.float32),
            jax.ShapeDtypeStruct((1, 1), jnp.float32),
        ],
        compiler_params=pltpu.CompilerParams(
            dimension_semantics=("arbitrary",),
            vmem_limit_bytes=100 * 1024 * 1024,
        ),
    )(xt, codebooks, cbt)
    return out_t.T, loss[0, 0]
